# baseline (speedup 1.0000x reference)
import functools

import jax
import jax.numpy as jnp
import numpy as np
from jax import lax
from jax.experimental import pallas as pl
from jax.experimental.pallas import tpu as pltpu
from jax.experimental.pallas import tpu_sc as plsc

_VOCAB = 100000
_D = 1024
_BATCH = 4
_SEQ = 2048
_N = _BATCH * _SEQ

_NC = 2
_NS = 16
_NW = _NC * _NS

_K = 4
_CW = _SEQ // _K
_NROWS_C = _BATCH * _CW
_BPW = _NROWS_C // _NW
_GR = 16
_NST = _BPW // _GR

_SCALE = float(np.sqrt(_D))


def _pe_table() -> np.ndarray:
    pos = np.arange(_SEQ, dtype=np.float64)[:, None]
    i = np.arange(0, _D, 2, dtype=np.float64)
    div = np.exp(-np.log(10000.0) * i / _D)
    pe = np.zeros((_SEQ, _D), dtype=np.float64)
    pe[:, 0::2] = np.sin(pos * div)
    pe[:, 1::2] = np.cos(pos * div)
    return pe.astype(np.float32)


_PE = _pe_table()


def _pe_dev():
    rows = 256

    def body(o_ref):
        i = pl.program_id(0)
        pos = (lax.broadcasted_iota(jnp.int32, (rows, _D), 0)
               + i * rows).astype(jnp.float32)
        col = lax.broadcasted_iota(jnp.int32, (rows, _D), 1)
        pair = (col // 2).astype(jnp.float32)
        ang = pos * jnp.exp(pair * (-2.0 * float(np.log(10000.0)) / _D))
        o_ref[...] = jnp.where(col % 2 == 0, jnp.sin(ang), jnp.cos(ang))

    return pl.pallas_call(
        body,
        grid=(_SEQ // rows,),
        out_specs=pl.BlockSpec((rows, _D), lambda i: (i, 0)),
        out_shape=jax.ShapeDtypeStruct((_SEQ, _D), jnp.float32),
    )()


_WPB = _NW // _BATCH
_PPW = _CW // _WPB


def _sc_gather(table, tok_flat, base):
    mesh = plsc.VectorSubcoreMesh(core_axis_name="c", subcore_axis_name="s")

    @functools.partial(
        pl.kernel,
        mesh=mesh,
        out_type=jax.ShapeDtypeStruct((_NROWS_C, _D), jnp.float32),
        scratch_types=[
            pltpu.VMEM((_BPW,), jnp.int32),
        ] + [pltpu.VMEM((_GR, _D), jnp.float32) for _ in range(_NST)]
          + [pltpu.SemaphoreType.DMA for _ in range(_NST)]
          + [pltpu.SemaphoreType.DMA],
    )
    def k(table_hbm, tok_hbm, out_hbm, idx_v, *rest):
        bufs = rest[:_NST]
        gsems = rest[_NST:2 * _NST]
        wsem = rest[2 * _NST]
        wid = lax.axis_index("s") * _NC + lax.axis_index("c")
        wbase = wid * _BPW
        src = (wid // _WPB) * _SEQ + base + (wid % _WPB) * _PPW
        pltpu.sync_copy(tok_hbm.at[pl.ds(src, _BPW)], idx_v)
        for j in range(_NST):
            pltpu.async_copy(
                table_hbm.at[idx_v.at[pl.ds(j * _GR, _GR)]], bufs[j], gsems[j])
        for j in range(_NST):
            pltpu.make_async_copy(
                table_hbm.at[pl.ds(0, _GR)], bufs[j], gsems[j]).wait()
            pltpu.async_copy(
                bufs[j], out_hbm.at[pl.ds(wbase + j * _GR, _GR)], wsem)
        for j in range(_NST):
            pltpu.make_async_copy(
                table_hbm.at[pl.ds(0, _GR)], bufs[0], wsem).wait()

    return k(table, tok_flat)


_POS_W = _SEQ // _NW
_FGR = 8
_FNS = _POS_W // _FGR
_GROWS = _BATCH * _FGR


def _sc_fused(table, tok_t, pe):
    mesh = plsc.VectorSubcoreMesh(core_axis_name="c", subcore_axis_name="s")

    @functools.partial(
        pl.kernel,
        mesh=mesh,
        out_type=jax.ShapeDtypeStruct((_N, _D), jnp.float32),
        scratch_types=[
            pltpu.VMEM((_BATCH * _POS_W,), jnp.int32),
        ] + [pltpu.VMEM((_GROWS, _D), jnp.float32) for _ in range(3)]
          + [pltpu.VMEM((_FGR, _D), jnp.float32) for _ in range(3)]
          + [pltpu.SemaphoreType.DMA for _ in range(7)],
    )
    def k(table_hbm, tok_hbm, pe_hbm, out_hbm, idx_v, *rest):
        gbufs = rest[0:3]
        pbufs = rest[3:6]
        gsems = rest[6:9]
        psems = rest[9:12]
        wsem = rest[12]
        wid = lax.axis_index("s") * _NC + lax.axis_index("c")
        pbase = wid * _POS_W
        for b in range(_BATCH):
            pltpu.sync_copy(tok_hbm.at[b, pl.ds(pbase, _POS_W)],
                            idx_v.at[pl.ds(b * _POS_W, _POS_W)])

        def _fire(s, m):
            for b in range(_BATCH):
                pltpu.async_copy(
                    table_hbm.at[idx_v.at[pl.ds(b * _POS_W + s * _FGR, _FGR)]],
                    gbufs[m].at[pl.ds(b * _FGR, _FGR)], gsems[m])
            pltpu.async_copy(
                pe_hbm.at[pl.ds(pbase + s * _FGR, _FGR)], pbufs[m], psems[m])

        def _step(s, m):
            g, p = gbufs[m], pbufs[m]
            pltpu.make_async_copy(
                table_hbm.at[pl.ds(0, _GROWS)], g, gsems[m]).wait()
            pltpu.make_async_copy(
                pe_hbm.at[pl.ds(0, _FGR)], p, psems[m]).wait()

            @pl.loop(0, _FGR)
            def _(r):
                for c in range(_D // 16):
                    cols = pl.ds(c * 16, 16)
                    pv = p.at[pl.ds(r, 1), cols][...]
                    gv = [g.at[pl.ds(r + 8 * b, 1), cols][...]
                          for b in range(_BATCH)]
                    for b in range(_BATCH):
                        g.at[pl.ds(r + 8 * b, 1), cols][...] = (
                            gv[b] * _SCALE + pv)

            @pl.when(s + 2 < _FNS)
            def _():
                @pl.when(s >= 1)
                def _():
                    pltpu.make_async_copy(
                        table_hbm.at[pl.ds(0, _GROWS)], g, wsem).wait()

                _fire(s + 2, (m + 2) % 3)

            for b in range(_BATCH):
                pltpu.async_copy(
                    g.at[pl.ds(b * _FGR, _FGR)],
                    out_hbm.at[pl.ds(b * _SEQ + pbase + s * _FGR, _FGR)],
                    wsem)

        _fire(0, 0)
        _fire(1, 1)

        @pl.loop(0, _FNS)
        def _(s):
            for mm in range(3):
                @pl.when(s % 3 == mm)
                def _(mm=mm):
                    _step(s, mm)

        for _ in range(3):
            pltpu.make_async_copy(
                table_hbm.at[pl.ds(0, _GROWS)], gbufs[0], wsem).wait()

    return k(table, tok_t, pe)


def _fixup_chunk(prev, gathered, pe, c):
    def body(*refs):
        g_ref, p_ref, o_ref = refs[-3], refs[-2], refs[-1]
        o_ref[...] = g_ref[...] * _SCALE + p_ref[...]

    in_specs = [
        pl.BlockSpec((_CW, _D), lambda b: (b, 0)),
        pl.BlockSpec((_CW, _D), lambda b: (c, 0)),
    ]
    operands = [gathered, pe]
    aliases = {}
    if prev is not None:
        in_specs = [pl.BlockSpec(memory_space=pl.ANY)] + in_specs
        operands = [prev] + operands
        aliases = {0: 0}

    return pl.pallas_call(
        body,
        grid=(_BATCH,),
        in_specs=in_specs,
        out_specs=pl.BlockSpec((_CW, _D), lambda b: (b * _K + c, 0)),
        out_shape=jax.ShapeDtypeStruct((_N, _D), jnp.float32),
        input_output_aliases=aliases,
    )(*operands)


def kernel(tokens, table):
    out = _sc_fused(table, tokens.astype(jnp.int32), _pe_dev())
    return out.reshape(_BATCH, _SEQ, _D)


# speedup vs baseline: 1.3940x; 1.3940x over previous
import functools

import jax
import jax.numpy as jnp
import numpy as np
from jax import lax
from jax.experimental import pallas as pl
from jax.experimental.pallas import tpu as pltpu
from jax.experimental.pallas import tpu_sc as plsc

_VOCAB = 100000
_D = 1024
_BATCH = 4
_SEQ = 2048
_N = _BATCH * _SEQ

_NC = 2
_NS = 16
_NW = _NC * _NS

_K = 4
_CW = _SEQ // _K
_NROWS_C = _BATCH * _CW
_BPW = _NROWS_C // _NW
_GR = 16
_NST = _BPW // _GR

_SCALE = float(np.sqrt(_D))


def _pe_table() -> np.ndarray:
    pos = np.arange(_SEQ, dtype=np.float64)[:, None]
    i = np.arange(0, _D, 2, dtype=np.float64)
    div = np.exp(-np.log(10000.0) * i / _D)
    pe = np.zeros((_SEQ, _D), dtype=np.float64)
    pe[:, 0::2] = np.sin(pos * div)
    pe[:, 1::2] = np.cos(pos * div)
    return pe.astype(np.float32)


_PE = _pe_table()


def _pe_dev():
    rows = 256

    def body(o_ref):
        i = pl.program_id(0)
        pos = (lax.broadcasted_iota(jnp.int32, (rows, _D), 0)
               + i * rows).astype(jnp.float32)
        col = lax.broadcasted_iota(jnp.int32, (rows, _D), 1)
        pair = (col // 2).astype(jnp.float32)
        ang = pos * jnp.exp(pair * (-2.0 * float(np.log(10000.0)) / _D))
        o_ref[...] = jnp.where(col % 2 == 0, jnp.sin(ang), jnp.cos(ang))

    return pl.pallas_call(
        body,
        grid=(_SEQ // rows,),
        out_specs=pl.BlockSpec((rows, _D), lambda i: (i, 0)),
        out_shape=jax.ShapeDtypeStruct((_SEQ, _D), jnp.float32),
    )()


_WPB = _NW // _BATCH
_PPW = _CW // _WPB


def _sc_gather(table, tok_flat, base):
    mesh = plsc.VectorSubcoreMesh(core_axis_name="c", subcore_axis_name="s")

    @functools.partial(
        pl.kernel,
        mesh=mesh,
        out_type=jax.ShapeDtypeStruct((_NROWS_C, _D), jnp.float32),
        scratch_types=[
            pltpu.VMEM((_BPW,), jnp.int32),
        ] + [pltpu.VMEM((_GR, _D), jnp.float32) for _ in range(_NST)]
          + [pltpu.SemaphoreType.DMA for _ in range(_NST)]
          + [pltpu.SemaphoreType.DMA],
    )
    def k(table_hbm, tok_hbm, out_hbm, idx_v, *rest):
        bufs = rest[:_NST]
        gsems = rest[_NST:2 * _NST]
        wsem = rest[2 * _NST]
        wid = lax.axis_index("s") * _NC + lax.axis_index("c")
        wbase = wid * _BPW
        src = (wid // _WPB) * _SEQ + base + (wid % _WPB) * _PPW
        pltpu.sync_copy(tok_hbm.at[pl.ds(src, _BPW)], idx_v)
        for j in range(_NST):
            pltpu.async_copy(
                table_hbm.at[idx_v.at[pl.ds(j * _GR, _GR)]], bufs[j], gsems[j])
        for j in range(_NST):
            pltpu.make_async_copy(
                table_hbm.at[pl.ds(0, _GR)], bufs[j], gsems[j]).wait()
            pltpu.async_copy(
                bufs[j], out_hbm.at[pl.ds(wbase + j * _GR, _GR)], wsem)
        for j in range(_NST):
            pltpu.make_async_copy(
                table_hbm.at[pl.ds(0, _GR)], bufs[0], wsem).wait()

    return k(table, tok_flat)


_POS_W = _SEQ // _NW
_FGR = 8
_FNS = _POS_W // _FGR
_GROWS = _BATCH * _FGR


def _sc_fused(table, tok_t, pe):
    mesh = plsc.VectorSubcoreMesh(core_axis_name="c", subcore_axis_name="s")

    @functools.partial(
        pl.kernel,
        mesh=mesh,
        out_type=jax.ShapeDtypeStruct((_N, _D), jnp.float32),
        scratch_types=[
            pltpu.VMEM((_BATCH * _POS_W,), jnp.int32),
        ] + [pltpu.VMEM((_GROWS, _D), jnp.float32) for _ in range(3)]
          + [pltpu.VMEM((_FGR, _D), jnp.float32) for _ in range(3)]
          + [pltpu.SemaphoreType.DMA for _ in range(7)],
    )
    def k(table_hbm, tok_hbm, pe_hbm, out_hbm, idx_v, *rest):
        gbufs = rest[0:3]
        pbufs = rest[3:6]
        gsems = rest[6:9]
        psems = rest[9:12]
        wsem = rest[12]
        wid = lax.axis_index("s") * _NC + lax.axis_index("c")
        pbase = wid * _POS_W
        for b in range(_BATCH):
            pltpu.sync_copy(tok_hbm.at[b, pl.ds(pbase, _POS_W)],
                            idx_v.at[pl.ds(b * _POS_W, _POS_W)])

        def _fire(s, m):
            for b in range(_BATCH):
                pltpu.async_copy(
                    table_hbm.at[idx_v.at[pl.ds(b * _POS_W + s * _FGR, _FGR)]],
                    gbufs[m].at[pl.ds(b * _FGR, _FGR)], gsems[m])
            pltpu.async_copy(
                pe_hbm.at[pl.ds(pbase + s * _FGR, _FGR)], pbufs[m], psems[m])

        def _step(s, m):
            g, p = gbufs[m], pbufs[m]
            pltpu.make_async_copy(
                table_hbm.at[pl.ds(0, _GROWS)], g, gsems[m]).wait()
            pltpu.make_async_copy(
                pe_hbm.at[pl.ds(0, _FGR)], p, psems[m]).wait()

            @pl.loop(0, _FGR)
            def _(r):
                for c in range(_D // 16):
                    cols = pl.ds(c * 16, 16)
                    pv = p.at[pl.ds(r, 1), cols][...]
                    gv = [g.at[pl.ds(r + 8 * b, 1), cols][...]
                          for b in range(_BATCH)]
                    for b in range(_BATCH):
                        g.at[pl.ds(r + 8 * b, 1), cols][...] = (
                            gv[b] * _SCALE + pv)

            @pl.when(s + 2 < _FNS)
            def _():
                @pl.when(s >= 1)
                def _():
                    pltpu.make_async_copy(
                        table_hbm.at[pl.ds(0, _GROWS)], g, wsem).wait()

                _fire(s + 2, (m + 2) % 3)

            for b in range(_BATCH):
                pltpu.async_copy(
                    g.at[pl.ds(b * _FGR, _FGR)],
                    out_hbm.at[pl.ds(b * _SEQ + pbase + s * _FGR, _FGR)],
                    wsem)

        _fire(0, 0)
        _fire(1, 1)

        @pl.loop(0, _FNS)
        def _(s):
            for mm in range(3):
                @pl.when(s % 3 == mm)
                def _(mm=mm):
                    _step(s, mm)

        for _ in range(3):
            pltpu.make_async_copy(
                table_hbm.at[pl.ds(0, _GROWS)], gbufs[0], wsem).wait()

    return k(table, tok_t, pe)


def _fixup_chunk(prev, gathered, pe, c):
    def body(*refs):
        g_ref, p_ref, o_ref = refs[-3], refs[-2], refs[-1]
        o_ref[...] = g_ref[...] * _SCALE + p_ref[...]

    in_specs = [
        pl.BlockSpec((_CW, _D), lambda b: (b, 0)),
        pl.BlockSpec((_CW, _D), lambda b: (c, 0)),
    ]
    operands = [gathered, pe]
    aliases = {}
    if prev is not None:
        in_specs = [pl.BlockSpec(memory_space=pl.ANY)] + in_specs
        operands = [prev] + operands
        aliases = {0: 0}

    return pl.pallas_call(
        body,
        grid=(_BATCH,),
        in_specs=in_specs,
        out_specs=pl.BlockSpec((_CW, _D), lambda b: (b * _K + c, 0)),
        out_shape=jax.ShapeDtypeStruct((_N, _D), jnp.float32),
        input_output_aliases=aliases,
    )(*operands)


def kernel(tokens, table):
    out = _sc_fused(table, tokens.astype(jnp.int32), jnp.asarray(_PE))
    return out.reshape(_BATCH, _SEQ, _D)
